# baseline (speedup 1.0000x reference)
import functools

import jax
import jax.numpy as jnp
from jax import lax
from jax.experimental import pallas as pl
from jax.experimental.pallas import tpu as pltpu
from jax.experimental.pallas import tpu_sc as plsc

N = 100000
J = 16
L = 16
NC = 2
NS = 16
W = NC * NS
GROUPS = N // L
GP_BASE = GROUPS // W
GP_EXTRA = GROUPS - GP_BASE * W
MAXG = GP_BASE + 1
MAXN = MAXG * L
G = 1024
ROW = 32
ACC = G * ROW
ACC_PER_SUB = ACC // NS
ENC_GRID = 10
ENC_B = N // ENC_GRID

LOG1P_C = (-0.01720806112146555, 0.08172680837613401, -0.18878267362211323,
           0.31459053537160714, -0.4969779111678143, 0.9997924357286277,
           3.507552052950621e-06)


def _enc_kernel(x_ref, y_ref, e_ref):
    x = x_ref[...]
    xb = x.astype(jnp.bfloat16)
    xi = lax.bitcast_convert_type(xb, jnp.uint16).astype(jnp.int32)
    yf = y_ref[...].astype(jnp.float32)
    wts = (jnp.int32(1) << lax.broadcasted_iota(jnp.int32, (J, 1), 0))
    p = jnp.dot(yf, wts.astype(jnp.float32))
    e_ref[...] = (xi.reshape(ENC_B, 1) << 16) | p.astype(jnp.int32)


def _fin_kernel(acc_ref, b_ref, o_ref):
    a = acc_ref[0] + acc_ref[1]
    s = a[:, 0:16]
    mx = jnp.max(s, axis=1, keepdims=True)
    f_sum = a[:, 16:17]
    cnt = a[:, 17:18]
    rows = lax.broadcasted_iota(jnp.int32, (G, 1), 0)
    ng = jnp.max(b_ref[...]) + 1
    val = jnp.where((cnt > 0.0) & (rows < ng),
                    (f_sum - mx) / jnp.maximum(cnt, 1.0), 0.0)
    o_ref[...] = jnp.full((1, 1), jnp.sum(val) / ng.astype(jnp.float32))


def _bcast_lane(vec, j):
    idx = jnp.full((L,), j, jnp.int32)
    return vec.at[idx].get(mode="promise_in_bounds")


def _sc_body(b_hbm, e_hbm, out_hbm,
             g_v, e_v, stg_v, idx_v, zb_v, acc_sh, sem):
    cid = lax.axis_index("c")
    sid = lax.axis_index("s")
    wid = cid * NS + sid

    gs = GP_BASE * wid + jnp.minimum(wid, GP_EXTRA)
    ngroups = jnp.where(wid < GP_EXTRA, GP_BASE + 1, GP_BASE)
    off = jnp.minimum(gs * L, N - MAXN)
    lo = gs * L - off

    c1 = pltpu.async_copy(b_hbm.at[pl.ds(off, MAXN)], g_v, sem)
    c2 = pltpu.async_copy(e_hbm.at[pl.ds(off, MAXN)], e_v, sem)

    li = lax.iota(jnp.int32, L)
    zf = jnp.zeros((L,), jnp.float32)

    def zero_zb(r, _):
        zb_v[pl.ds(r * L, L)] = zf
        return 0

    lax.fori_loop(0, ACC_PER_SUB // L, zero_zb, 0)

    pltpu.sync_copy(zb_v, acc_sh.at[pl.ds(sid * ACC_PER_SUB, ACC_PER_SUB)])
    c1.wait()
    c2.wait()
    plsc.subcore_barrier()

    lane0 = li == 0
    lane1 = li == 1
    zeros = jnp.zeros((L,), jnp.float32)
    bitmask = jnp.int32(1) << li
    zero_i = jnp.zeros((L,), jnp.int32)
    himask = jnp.full((L,), jnp.int32(-65536))

    prev_g0 = g_v[pl.ds(lo, L)][0]

    def do_flush(pg, acc_s, acc_fv, cnt):
        base = pg * ROW
        sf = jnp.sum(acc_fv)
        fc = jnp.where(lane0, jnp.full((L,), sf), zeros) \
           + jnp.where(lane1, jnp.full((L,), cnt.astype(jnp.float32)), zeros)
        stg_v[pl.ds(0, L)] = acc_s
        stg_v[pl.ds(L, L)] = fc
        idx_v[pl.ds(0, L)] = base + li
        idx_v[pl.ds(L, L)] = base + L + li
        pltpu.sync_copy(stg_v, acc_sh.at[idx_v], add=True)

    def body(i, carry):
        prev_g, acc_s, acc_fv, cnt = carry
        nb = lo + i * L
        gvi = g_v[pl.ds(nb, L)]
        ev = e_v[pl.ds(nb, L)]
        xv = lax.bitcast_convert_type(ev & himask, jnp.float32)
        g_first = gvi[0]
        g_last = gvi[15]

        t = jnp.exp(-jnp.abs(xv))
        pf = jnp.full((L,), LOG1P_C[0])
        for coef in LOG1P_C[1:]:
            pf = pf * t + coef
        f_vec = jnp.maximum(xv, zeros) + pf

        def fast(args):
            prev_g, acc_s, acc_fv, cnt = args
            changed = g_first != prev_g

            @pl.when(changed)
            def _():
                do_flush(prev_g, acc_s, acc_fv, cnt)

            pvb = jnp.full((L,), changed)
            acc_s = jnp.where(pvb, zeros, acc_s)
            acc_fv = jnp.where(pvb, zeros, acc_fv)
            cnt = jnp.where(changed, 0, cnt)
            acc_b = zeros
            for j in range(0, L, 2):
                pb = _bcast_lane(ev, j)
                m = (pb & bitmask) != zero_i
                xb = _bcast_lane(xv, j)
                acc_s = jnp.where(m, acc_s + xb, acc_s)
                pb2 = _bcast_lane(ev, j + 1)
                m2 = (pb2 & bitmask) != zero_i
                xb2 = _bcast_lane(xv, j + 1)
                acc_b = jnp.where(m2, acc_b + xb2, acc_b)
            acc_s = acc_s + acc_b
            acc_fv = acc_fv + f_vec
            cnt = cnt + L
            return g_first, acc_s, acc_fv, cnt

        def slow(args):
            prev_g, acc_s, acc_fv, cnt = args
            for j in range(L):
                gj = gvi[j]
                changed = gj != prev_g

                @pl.when(changed)
                def _(pgx=prev_g, asx=acc_s, afx=acc_fv, cnx=cnt):
                    do_flush(pgx, asx, afx, cnx)

                pvb = jnp.full((L,), changed)
                acc_s = jnp.where(pvb, zeros, acc_s)
                acc_fv = jnp.where(pvb, zeros, acc_fv)
                cnt = jnp.where(changed, 0, cnt)
                pb = _bcast_lane(ev, j)
                m = (pb & bitmask) != zero_i
                xb = _bcast_lane(xv, j)
                acc_s = jnp.where(m, acc_s + xb, acc_s)
                acc_fv = acc_fv + jnp.where(li == j, f_vec, zeros)
                cnt = cnt + 1
                prev_g = jnp.where(changed, gj, prev_g)
            return prev_g, acc_s, acc_fv, cnt

        return lax.cond(g_first == g_last, fast, slow,
                        (prev_g, acc_s, acc_fv, cnt))

    prev_g, acc_s, acc_fv, cnt = lax.fori_loop(
        0, ngroups, body, (prev_g0, zeros, zeros, jnp.int32(0)))
    do_flush(prev_g, acc_s, acc_fv, cnt)
    plsc.subcore_barrier()

    pltpu.sync_copy(acc_sh.at[pl.ds(sid * ACC_PER_SUB, ACC_PER_SUB)],
                    out_hbm.at[cid, pl.ds(sid * ACC_PER_SUB, ACC_PER_SUB)])


@functools.partial(
    pl.kernel,
    out_type=jax.ShapeDtypeStruct((NC, ACC), jnp.float32),
    mesh=plsc.VectorSubcoreMesh(core_axis_name="c", subcore_axis_name="s"),
    compiler_params=pltpu.CompilerParams(needs_layout_passes=False),
    scratch_types=[
        pltpu.VMEM((MAXN,), jnp.int32),
        pltpu.VMEM((MAXN,), jnp.int32),
        pltpu.VMEM((2 * L,), jnp.float32),
        pltpu.VMEM((2 * L,), jnp.int32),
        pltpu.VMEM((ACC_PER_SUB,), jnp.float32),
        pltpu.VMEM_SHARED((ACC,), jnp.float32),
        pltpu.SemaphoreType.DMA,
    ],
)
def _sc_call(b_hbm, e_hbm, out_hbm,
             g_v, e_v, stg_v, idx_v, zb_v, acc_sh, sem):
    _sc_body(b_hbm, e_hbm, out_hbm,
             g_v, e_v, stg_v, idx_v, zb_v, acc_sh, sem)


def kernel(logits, y, batch):
    yi = y.astype(jnp.int32)
    bi = batch.astype(jnp.int32)

    e2d = pl.pallas_call(
        _enc_kernel,
        grid=(ENC_GRID,),
        in_specs=[
            pl.BlockSpec((ENC_B, 1), lambda i: (i, 0)),
            pl.BlockSpec((ENC_B, J), lambda i: (i, 0)),
        ],
        out_specs=pl.BlockSpec((ENC_B, 1), lambda i: (i, 0)),
        out_shape=jax.ShapeDtypeStruct((N, 1), jnp.int32),
    )(logits.astype(jnp.float32), yi)
    e = e2d.reshape(N)

    acc = _sc_call(bi, e).reshape(NC, G, ROW)

    res = pl.pallas_call(
        _fin_kernel,
        out_shape=jax.ShapeDtypeStruct((1, 1), jnp.float32),
    )(acc, bi.reshape(800, 125))
    return res[0, 0]


# speedup vs baseline: 1.6881x; 1.6881x over previous
import functools

import jax
import jax.numpy as jnp
from jax import lax
from jax.experimental import pallas as pl
from jax.experimental.pallas import tpu as pltpu
from jax.experimental.pallas import tpu_sc as plsc

N = 100000
J = 16
L = 16
NC = 2
NS = 16
W = NC * NS
GROUPS = N // L
GP_BASE = GROUPS // W
GP_EXTRA = GROUPS - GP_BASE * W
MAXG = GP_BASE + 1
MAXN = MAXG * L
G = 1024
ROW = 32
ACC = G * ROW
ACC_PER_SUB = ACC // NS
NG_SLOT = 18

LOG1P_C = (-0.01720806112146555, 0.08172680837613401, -0.18878267362211323,
           0.31459053537160714, -0.4969779111678143, 0.9997924357286277,
           3.507552052950621e-06)


def _fin_kernel(acc_ref, o_ref):
    a = acc_ref[0] + acc_ref[1]
    s = a[:, 0:16]
    mx = jnp.max(s, axis=1, keepdims=True)
    f_sum = a[:, 16:17]
    cnt = a[:, 17:18]
    rows = lax.broadcasted_iota(jnp.int32, (G, 1), 0)
    ng_f = jnp.sum(a[0:1, NG_SLOT:NG_SLOT + 1])
    ng = ng_f.astype(jnp.int32)
    val = jnp.where((cnt > 0.0) & (rows < ng),
                    (f_sum - mx) / jnp.maximum(cnt, 1.0), 0.0)
    o_ref[...] = jnp.full((1, 1), jnp.sum(val) / ng_f)


def _bcast_lane(vec, j):
    idx = jnp.full((L,), j, jnp.int32)
    return vec.at[idx].get(mode="promise_in_bounds")


def _sc_body(x_hbm, b_hbm, y_hbm, out_hbm,
             x_v, g_v, y_v, stg_v, idx_v, zb_v, acc_sh, sem):
    cid = lax.axis_index("c")
    sid = lax.axis_index("s")
    wid = cid * NS + sid

    gs = GP_BASE * wid + jnp.minimum(wid, GP_EXTRA)
    ngroups = jnp.where(wid < GP_EXTRA, GP_BASE + 1, GP_BASE)
    off = jnp.minimum(gs * L, N - MAXN)
    lo = gs * L - off

    c1 = pltpu.async_copy(x_hbm.at[pl.ds(off, MAXN)], x_v, sem)
    c2 = pltpu.async_copy(b_hbm.at[pl.ds(off, MAXN)], g_v, sem)
    c3 = pltpu.async_copy(y_hbm.at[pl.ds(off * J, MAXN * J)], y_v, sem)

    li = lax.iota(jnp.int32, L)
    zf = jnp.zeros((L,), jnp.float32)

    def zero_zb(r, _):
        zb_v[pl.ds(r * L, L)] = zf
        return 0

    lax.fori_loop(0, ACC_PER_SUB // L, zero_zb, 0)

    pltpu.sync_copy(zb_v, acc_sh.at[pl.ds(sid * ACC_PER_SUB, ACC_PER_SUB)])
    c1.wait()
    c2.wait()
    c3.wait()
    plsc.subcore_barrier()

    lane0 = li == 0
    lane1 = li == 1
    zeros = jnp.zeros((L,), jnp.float32)

    prev_g0 = g_v[pl.ds(lo, L)][0]

    def do_flush(pg, acc_s, acc_fv, cnt):
        base = pg * ROW
        sf = jnp.sum(acc_fv)
        fc = jnp.where(lane0, jnp.full((L,), sf), zeros) \
           + jnp.where(lane1, jnp.full((L,), cnt.astype(jnp.float32)), zeros)
        stg_v[pl.ds(0, L)] = acc_s
        stg_v[pl.ds(L, L)] = fc
        idx_v[pl.ds(0, L)] = base + li
        idx_v[pl.ds(L, L)] = base + L + li
        pltpu.sync_copy(stg_v, acc_sh.at[idx_v], add=True)

    def body(i, carry):
        prev_g, acc_s, acc_fv, cnt = carry
        nb = lo + i * L
        gvi = g_v[pl.ds(nb, L)]
        xv = x_v[pl.ds(nb, L)]
        g_first = gvi[0]
        g_last = gvi[15]

        t = jnp.exp(-jnp.abs(xv))
        pf = jnp.full((L,), LOG1P_C[0])
        for coef in LOG1P_C[1:]:
            pf = pf * t + coef
        f_vec = jnp.maximum(xv, zeros) + pf

        def fast(args):
            prev_g, acc_s, acc_fv, cnt = args
            changed = g_first != prev_g

            @pl.when(changed)
            def _():
                do_flush(prev_g, acc_s, acc_fv, cnt)

            pvb = jnp.full((L,), changed)
            acc_s = jnp.where(pvb, zeros, acc_s)
            acc_fv = jnp.where(pvb, zeros, acc_fv)
            cnt = jnp.where(changed, 0, cnt)
            acc_b = zeros
            for j in range(0, L, 2):
                ycv = y_v[pl.ds((nb + j) * J, L)].astype(jnp.float32)
                acc_s = acc_s + _bcast_lane(xv, j) * ycv
                ycv2 = y_v[pl.ds((nb + j + 1) * J, L)].astype(jnp.float32)
                acc_b = acc_b + _bcast_lane(xv, j + 1) * ycv2
            acc_s = acc_s + acc_b
            acc_fv = acc_fv + f_vec
            cnt = cnt + L
            return g_first, acc_s, acc_fv, cnt

        def slow(args):
            prev_g, acc_s, acc_fv, cnt = args
            for j in range(L):
                gj = gvi[j]
                changed = gj != prev_g

                @pl.when(changed)
                def _(pgx=prev_g, asx=acc_s, afx=acc_fv, cnx=cnt):
                    do_flush(pgx, asx, afx, cnx)

                pvb = jnp.full((L,), changed)
                acc_s = jnp.where(pvb, zeros, acc_s)
                acc_fv = jnp.where(pvb, zeros, acc_fv)
                cnt = jnp.where(changed, 0, cnt)
                ycv = y_v[pl.ds((nb + j) * J, L)].astype(jnp.float32)
                acc_s = acc_s + _bcast_lane(xv, j) * ycv
                acc_fv = acc_fv + jnp.where(li == j, f_vec, zeros)
                cnt = cnt + 1
                prev_g = jnp.where(changed, gj, prev_g)
            return prev_g, acc_s, acc_fv, cnt

        return lax.cond(g_first == g_last, fast, slow,
                        (prev_g, acc_s, acc_fv, cnt))

    prev_g, acc_s, acc_fv, cnt = lax.fori_loop(
        0, ngroups, body, (prev_g0, zeros, zeros, jnp.int32(0)))
    do_flush(prev_g, acc_s, acc_fv, cnt)

    @pl.when(wid == W - 1)
    def _():
        ngf = (prev_g + 1).astype(jnp.float32)
        stg_v[pl.ds(0, L)] = jnp.where(lane0, jnp.full((L,), ngf), zeros)
        stg_v[pl.ds(L, L)] = zeros
        idx_v[pl.ds(0, L)] = NG_SLOT + li
        idx_v[pl.ds(L, L)] = NG_SLOT + L + li
        pltpu.sync_copy(stg_v, acc_sh.at[idx_v], add=True)

    plsc.subcore_barrier()

    pltpu.sync_copy(acc_sh.at[pl.ds(sid * ACC_PER_SUB, ACC_PER_SUB)],
                    out_hbm.at[cid, pl.ds(sid * ACC_PER_SUB, ACC_PER_SUB)])


@functools.partial(
    pl.kernel,
    out_type=jax.ShapeDtypeStruct((NC, ACC), jnp.float32),
    mesh=plsc.VectorSubcoreMesh(core_axis_name="c", subcore_axis_name="s"),
    compiler_params=pltpu.CompilerParams(needs_layout_passes=False),
    scratch_types=[
        pltpu.VMEM((MAXN,), jnp.float32),
        pltpu.VMEM((MAXN,), jnp.int32),
        pltpu.VMEM((MAXN * J,), jnp.int32),
        pltpu.VMEM((2 * L,), jnp.float32),
        pltpu.VMEM((2 * L,), jnp.int32),
        pltpu.VMEM((ACC_PER_SUB,), jnp.float32),
        pltpu.VMEM_SHARED((ACC,), jnp.float32),
        pltpu.SemaphoreType.DMA,
    ],
)
def _sc_call(x_hbm, b_hbm, y_hbm, out_hbm,
             x_v, g_v, y_v, stg_v, idx_v, zb_v, acc_sh, sem):
    _sc_body(x_hbm, b_hbm, y_hbm, out_hbm,
             x_v, g_v, y_v, stg_v, idx_v, zb_v, acc_sh, sem)


def kernel(logits, y, batch):
    x = logits.astype(jnp.float32).reshape(N)
    yi = y.astype(jnp.int32).reshape(N * J)
    bi = batch.astype(jnp.int32)

    acc = _sc_call(x, bi, yi).reshape(NC, G, ROW)

    res = pl.pallas_call(
        _fin_kernel,
        out_shape=jax.ShapeDtypeStruct((1, 1), jnp.float32),
    )(acc)
    return res[0, 0]


# speedup vs baseline: 1.7010x; 1.0077x over previous
import functools

import jax
import jax.numpy as jnp
from jax import lax
from jax.experimental import pallas as pl
from jax.experimental.pallas import tpu as pltpu
from jax.experimental.pallas import tpu_sc as plsc

N = 100000
J = 16
L = 16
NC = 2
NS = 16
W = NC * NS
GROUPS = N // L
GP_BASE = GROUPS // W
GP_EXTRA = GROUPS - GP_BASE * W
MAXG = GP_BASE + 1
MAXN = MAXG * L
G = 1024
ROW = 32
ACC = G * ROW
ACC_PER_SUB = ACC // NS
NG_SLOT = 18

LOG1P_C = (-0.01720806112146555, 0.08172680837613401, -0.18878267362211323,
           0.31459053537160714, -0.4969779111678143, 0.9997924357286277,
           3.507552052950621e-06)


def _fin_kernel(acc_ref, o_ref):
    a = acc_ref[0] + acc_ref[1]
    s = a[:, 0:16]
    mx = jnp.max(s, axis=1, keepdims=True)
    f_sum = a[:, 16:17]
    cnt = a[:, 17:18]
    rows = lax.broadcasted_iota(jnp.int32, (G, 1), 0)
    ng_f = jnp.sum(a[0:1, NG_SLOT:NG_SLOT + 1])
    ng = ng_f.astype(jnp.int32)
    val = jnp.where((cnt > 0.0) & (rows < ng),
                    (f_sum - mx) / jnp.maximum(cnt, 1.0), 0.0)
    o_ref[...] = jnp.full((1, 1), jnp.sum(val) / ng_f)


def _bcast_lane(vec, j):
    idx = jnp.full((L,), j, jnp.int32)
    return vec.at[idx].get(mode="promise_in_bounds")


def _sc_body(x_hbm, b_hbm, y_hbm, out_hbm,
             x_v, g_v, y_v, stg_v, idx_v, zb_v, acc_sh, sem):
    cid = lax.axis_index("c")
    sid = lax.axis_index("s")
    wid = cid * NS + sid

    gs = GP_BASE * wid + jnp.minimum(wid, GP_EXTRA)
    ngroups = jnp.where(wid < GP_EXTRA, GP_BASE + 1, GP_BASE)
    off = jnp.minimum(gs * L, N - MAXN)
    lo = gs * L - off

    c1 = pltpu.async_copy(x_hbm.at[pl.ds(off, MAXN)], x_v, sem)
    c2 = pltpu.async_copy(b_hbm.at[pl.ds(off, MAXN)], g_v, sem)
    c3 = pltpu.async_copy(y_hbm.at[pl.ds(off * J, MAXN * J)], y_v, sem)

    li = lax.iota(jnp.int32, L)
    zf = jnp.zeros((L,), jnp.float32)

    def zero_zb(r, _):
        zb_v[pl.ds(r * L, L)] = zf
        return 0

    lax.fori_loop(0, ACC_PER_SUB // L, zero_zb, 0)

    pltpu.sync_copy(zb_v, acc_sh.at[pl.ds(sid * ACC_PER_SUB, ACC_PER_SUB)])
    c1.wait()
    c2.wait()
    c3.wait()
    plsc.subcore_barrier()

    lane0 = li == 0
    lane1 = li == 1
    zeros = jnp.zeros((L,), jnp.float32)

    prev_g0 = g_v[pl.ds(lo, L)][0]

    def do_flush(pg, acc_s, acc_fv, cnt):
        base = pg * ROW
        sf = jnp.sum(acc_fv)
        fc = jnp.where(lane0, jnp.full((L,), sf), zeros) \
           + jnp.where(lane1, jnp.full((L,), cnt.astype(jnp.float32)), zeros)
        stg_v[pl.ds(0, L)] = acc_s
        stg_v[pl.ds(L, L)] = fc
        idx_v[pl.ds(0, L)] = base + li
        idx_v[pl.ds(L, L)] = base + L + li
        pltpu.sync_copy(stg_v, acc_sh.at[idx_v], add=True)

    def body(i, carry):
        prev_g, acc_s, acc_fv, cnt = carry
        nb = lo + i * L
        gvi = g_v[pl.ds(nb, L)]
        xv = x_v[pl.ds(nb, L)]
        g_first = gvi[0]
        g_last = gvi[15]

        t = jnp.exp(-jnp.abs(xv))
        pf = jnp.full((L,), LOG1P_C[0])
        for coef in LOG1P_C[1:]:
            pf = pf * t + coef
        f_vec = jnp.maximum(xv, zeros) + pf

        def fast(args):
            prev_g, acc_s, acc_fv, cnt = args
            changed = g_first != prev_g

            @pl.when(changed)
            def _():
                do_flush(prev_g, acc_s, acc_fv, cnt)

            pvb = jnp.full((L,), changed)
            acc_s = jnp.where(pvb, zeros, acc_s)
            acc_fv = jnp.where(pvb, zeros, acc_fv)
            cnt = jnp.where(changed, 0, cnt)
            acc_b = zeros
            for j in range(0, L, 2):
                m = y_v[pl.ds((nb + j) * J, L)] != 0
                acc_s = jnp.where(m, acc_s + _bcast_lane(xv, j), acc_s)
                m2 = y_v[pl.ds((nb + j + 1) * J, L)] != 0
                acc_b = jnp.where(m2, acc_b + _bcast_lane(xv, j + 1), acc_b)
            acc_s = acc_s + acc_b
            acc_fv = acc_fv + f_vec
            cnt = cnt + L
            return g_first, acc_s, acc_fv, cnt

        def slow(args):
            prev_g, acc_s, acc_fv, cnt = args
            for j in range(L):
                gj = gvi[j]
                changed = gj != prev_g

                @pl.when(changed)
                def _(pgx=prev_g, asx=acc_s, afx=acc_fv, cnx=cnt):
                    do_flush(pgx, asx, afx, cnx)

                pvb = jnp.full((L,), changed)
                acc_s = jnp.where(pvb, zeros, acc_s)
                acc_fv = jnp.where(pvb, zeros, acc_fv)
                cnt = jnp.where(changed, 0, cnt)
                m = y_v[pl.ds((nb + j) * J, L)] != 0
                acc_s = jnp.where(m, acc_s + _bcast_lane(xv, j), acc_s)
                acc_fv = acc_fv + jnp.where(li == j, f_vec, zeros)
                cnt = cnt + 1
                prev_g = jnp.where(changed, gj, prev_g)
            return prev_g, acc_s, acc_fv, cnt

        return lax.cond(g_first == g_last, fast, slow,
                        (prev_g, acc_s, acc_fv, cnt))

    prev_g, acc_s, acc_fv, cnt = lax.fori_loop(
        0, ngroups, body, (prev_g0, zeros, zeros, jnp.int32(0)))
    do_flush(prev_g, acc_s, acc_fv, cnt)

    @pl.when(wid == W - 1)
    def _():
        ngf = (prev_g + 1).astype(jnp.float32)
        stg_v[pl.ds(0, L)] = jnp.where(lane0, jnp.full((L,), ngf), zeros)
        stg_v[pl.ds(L, L)] = zeros
        idx_v[pl.ds(0, L)] = NG_SLOT + li
        idx_v[pl.ds(L, L)] = NG_SLOT + L + li
        pltpu.sync_copy(stg_v, acc_sh.at[idx_v], add=True)

    plsc.subcore_barrier()

    pltpu.sync_copy(acc_sh.at[pl.ds(sid * ACC_PER_SUB, ACC_PER_SUB)],
                    out_hbm.at[cid, pl.ds(sid * ACC_PER_SUB, ACC_PER_SUB)])


@functools.partial(
    pl.kernel,
    out_type=jax.ShapeDtypeStruct((NC, ACC), jnp.float32),
    mesh=plsc.VectorSubcoreMesh(core_axis_name="c", subcore_axis_name="s"),
    compiler_params=pltpu.CompilerParams(needs_layout_passes=False),
    scratch_types=[
        pltpu.VMEM((MAXN,), jnp.float32),
        pltpu.VMEM((MAXN,), jnp.int32),
        pltpu.VMEM((MAXN * J,), jnp.int32),
        pltpu.VMEM((2 * L,), jnp.float32),
        pltpu.VMEM((2 * L,), jnp.int32),
        pltpu.VMEM((ACC_PER_SUB,), jnp.float32),
        pltpu.VMEM_SHARED((ACC,), jnp.float32),
        pltpu.SemaphoreType.DMA,
    ],
)
def _sc_call(x_hbm, b_hbm, y_hbm, out_hbm,
             x_v, g_v, y_v, stg_v, idx_v, zb_v, acc_sh, sem):
    _sc_body(x_hbm, b_hbm, y_hbm, out_hbm,
             x_v, g_v, y_v, stg_v, idx_v, zb_v, acc_sh, sem)


def kernel(logits, y, batch):
    x = logits.astype(jnp.float32).reshape(N)
    yi = y.astype(jnp.int32).reshape(N * J)
    bi = batch.astype(jnp.int32)

    acc = _sc_call(x, bi, yi).reshape(NC, G, ROW)

    res = pl.pallas_call(
        _fin_kernel,
        out_shape=jax.ShapeDtypeStruct((1, 1), jnp.float32),
    )(acc)
    return res[0, 0]


# speedup vs baseline: 1.7036x; 1.0015x over previous
import functools

import jax
import jax.numpy as jnp
from jax import lax
from jax.experimental import pallas as pl
from jax.experimental.pallas import tpu as pltpu
from jax.experimental.pallas import tpu_sc as plsc

N = 100000
J = 16
L = 16
NC = 2
NS = 16
W = NC * NS
GROUPS = N // L
GP_BASE = GROUPS // W
GP_EXTRA = GROUPS - GP_BASE * W
MAXG = GP_BASE + 1
MAXN = MAXG * L
G = 1024
ROW = 32
ACC = G * ROW
ACC_PER_SUB = ACC // NS
NG_SLOT = 18

LOG1P_C = (-0.01720806112146555, 0.08172680837613401, -0.18878267362211323,
           0.31459053537160714, -0.4969779111678143, 0.9997924357286277,
           3.507552052950621e-06)


def _fin_kernel(acc_ref, o_ref):
    a = acc_ref[0] + acc_ref[1]
    s = a[:, 0:16]
    mx = jnp.max(s, axis=1, keepdims=True)
    f_sum = a[:, 16:17]
    cnt = a[:, 17:18]
    rows = lax.broadcasted_iota(jnp.int32, (G, 1), 0)
    ng_f = jnp.sum(a[0:1, NG_SLOT:NG_SLOT + 1])
    ng = ng_f.astype(jnp.int32)
    val = jnp.where((cnt > 0.0) & (rows < ng),
                    (f_sum - mx) / jnp.maximum(cnt, 1.0), 0.0)
    o_ref[...] = jnp.full((1, 1), jnp.sum(val) / ng_f)


def _bcast_lane(vec, j):
    idx = jnp.full((L,), j, jnp.int32)
    return vec.at[idx].get(mode="promise_in_bounds")


def _sc_body(x_hbm, b_hbm, y_hbm, out_hbm,
             x_v, g_v, y_v, stg_v, idx_v, zb_v, acc_sh, sem):
    cid = lax.axis_index("c")
    sid = lax.axis_index("s")
    wid = cid * NS + sid

    gs = GP_BASE * wid + jnp.minimum(wid, GP_EXTRA)
    ngroups = jnp.where(wid < GP_EXTRA, GP_BASE + 1, GP_BASE)
    off = jnp.minimum(gs * L, N - MAXN)
    lo = gs * L - off

    c1 = pltpu.async_copy(x_hbm.at[pl.ds(off, MAXN)], x_v, sem)
    c2 = pltpu.async_copy(b_hbm.at[pl.ds(off, MAXN)], g_v, sem)
    c3 = pltpu.async_copy(y_hbm.at[pl.ds(off * J, MAXN * J)], y_v, sem)

    li = lax.iota(jnp.int32, L)
    zf = jnp.zeros((L,), jnp.float32)

    def zero_zb(r, _):
        zb_v[pl.ds(r * L, L)] = zf
        return 0

    lax.fori_loop(0, ACC_PER_SUB // L, zero_zb, 0)

    pltpu.sync_copy(zb_v, acc_sh.at[pl.ds(sid * ACC_PER_SUB, ACC_PER_SUB)])
    c1.wait()
    c2.wait()
    c3.wait()
    plsc.subcore_barrier()

    lane0 = li == 0
    lane1 = li == 1
    zeros = jnp.zeros((L,), jnp.float32)

    prev_g0 = g_v[pl.ds(lo, L)][0]

    def do_flush(pg, acc_s, acc_fv, cnt):
        base = pg * ROW
        sf = jnp.sum(acc_fv)
        fc = jnp.where(lane0, jnp.full((L,), sf), zeros) \
           + jnp.where(lane1, jnp.full((L,), cnt.astype(jnp.float32)), zeros)
        stg_v[pl.ds(0, L)] = acc_s
        stg_v[pl.ds(L, L)] = fc
        idx_v[pl.ds(0, L)] = base + li
        idx_v[pl.ds(L, L)] = base + L + li
        pltpu.sync_copy(stg_v, acc_sh.at[idx_v], add=True)

    def body(i, carry):
        prev_g, acc_s, acc_fv, cnt = carry
        nb = lo + i * L
        gvi = g_v[pl.ds(nb, L)]
        xv = x_v[pl.ds(nb, L)]
        g_first = gvi[0]
        g_last = gvi[15]

        t = jnp.exp(-jnp.abs(xv))
        pf = jnp.full((L,), LOG1P_C[0])
        for coef in LOG1P_C[1:]:
            pf = pf * t + coef
        f_vec = jnp.maximum(xv, zeros) + pf

        def fast(args):
            prev_g, acc_s, acc_fv, cnt = args
            changed = g_first != prev_g

            @pl.when(changed)
            def _():
                do_flush(prev_g, acc_s, acc_fv, cnt)

            pvb = jnp.full((L,), changed)
            acc_s = jnp.where(pvb, zeros, acc_s)
            acc_fv = jnp.where(pvb, zeros, acc_fv)
            cnt = jnp.where(changed, 0, cnt)
            acc_b = zeros
            for j in range(0, L, 2):
                m = y_v[pl.ds((nb + j) * J, L)] != 0
                acc_s = jnp.where(m, acc_s + _bcast_lane(xv, j), acc_s)
                m2 = y_v[pl.ds((nb + j + 1) * J, L)] != 0
                acc_b = jnp.where(m2, acc_b + _bcast_lane(xv, j + 1), acc_b)
            acc_s = acc_s + acc_b
            acc_fv = acc_fv + f_vec
            cnt = cnt + L
            return g_first, acc_s, acc_fv, cnt

        def slow(args):
            prev_g, acc_s, acc_fv, cnt = args
            for j in range(L):
                gj = gvi[j]
                changed = gj != prev_g

                @pl.when(changed)
                def _(pgx=prev_g, asx=acc_s, afx=acc_fv, cnx=cnt):
                    do_flush(pgx, asx, afx, cnx)

                pvb = jnp.full((L,), changed)
                acc_s = jnp.where(pvb, zeros, acc_s)
                acc_fv = jnp.where(pvb, zeros, acc_fv)
                cnt = jnp.where(changed, 0, cnt)
                m = y_v[pl.ds((nb + j) * J, L)] != 0
                acc_s = jnp.where(m, acc_s + _bcast_lane(xv, j), acc_s)
                acc_fv = acc_fv + jnp.where(li == j, f_vec, zeros)
                cnt = cnt + 1
                prev_g = jnp.where(changed, gj, prev_g)
            return prev_g, acc_s, acc_fv, cnt

        return lax.cond(g_first == g_last, fast, slow,
                        (prev_g, acc_s, acc_fv, cnt))

    prev_g, acc_s, acc_fv, cnt = lax.fori_loop(
        0, ngroups, body, (prev_g0, zeros, zeros, jnp.int32(0)))
    do_flush(prev_g, acc_s, acc_fv, cnt)

    @pl.when(wid == W - 1)
    def _():
        ngf = (prev_g + 1).astype(jnp.float32)
        stg_v[pl.ds(0, L)] = jnp.where(lane0, jnp.full((L,), ngf), zeros)
        stg_v[pl.ds(L, L)] = zeros
        idx_v[pl.ds(0, L)] = NG_SLOT + li
        idx_v[pl.ds(L, L)] = NG_SLOT + L + li
        pltpu.sync_copy(stg_v, acc_sh.at[idx_v], add=True)

    plsc.subcore_barrier()

    pltpu.sync_copy(acc_sh.at[pl.ds(sid * ACC_PER_SUB, ACC_PER_SUB)],
                    out_hbm.at[cid, pl.ds(sid * ACC_PER_SUB, ACC_PER_SUB)])


@functools.partial(
    pl.kernel,
    out_type=jax.ShapeDtypeStruct((NC, ACC), jnp.float32),
    mesh=plsc.VectorSubcoreMesh(core_axis_name="c", subcore_axis_name="s"),
    compiler_params=pltpu.CompilerParams(needs_layout_passes=False),
    scratch_types=[
        pltpu.VMEM((MAXN,), jnp.float32),
        pltpu.VMEM((MAXN,), jnp.int32),
        pltpu.VMEM((MAXN * J,), jnp.int32),
        pltpu.VMEM((2 * L,), jnp.float32),
        pltpu.VMEM((2 * L,), jnp.int32),
        pltpu.VMEM((ACC_PER_SUB,), jnp.float32),
        pltpu.VMEM_SHARED((ACC,), jnp.float32),
        pltpu.SemaphoreType.DMA,
    ],
)
def _sc_call(x_hbm, b_hbm, y_hbm, out_hbm,
             x_v, g_v, y_v, stg_v, idx_v, zb_v, acc_sh, sem):
    _sc_body(x_hbm, b_hbm, y_hbm, out_hbm,
             x_v, g_v, y_v, stg_v, idx_v, zb_v, acc_sh, sem)


def kernel(logits, y, batch):
    x = lax.squeeze(logits.astype(jnp.float32), (1,))
    yi = y.astype(jnp.int32).reshape(N * J)
    bi = batch.astype(jnp.int32)

    acc = _sc_call(x, bi, yi).reshape(NC, G, ROW)

    res = pl.pallas_call(
        _fin_kernel,
        out_shape=jax.ShapeDtypeStruct((1, 1), jnp.float32),
    )(acc)
    return res[0, 0]
